# Initial kernel scaffold; baseline (speedup 1.0000x reference)
import jax
import jax.numpy as jnp
from jax.experimental import pallas as pl


def kernel(street_idxs, table):
    raise NotImplementedError("write your pallas kernel here")


# speedup vs baseline: 19.9152x; 19.9152x over previous
import jax
import jax.numpy as jnp
from jax.experimental import pallas as pl

_NUM_STREETS = 4
_EMBED_DIM = 128
_MAX_SEQ_LEN = 128
_VOCAB = _NUM_STREETS + 3
_PAD_TOKEN = _NUM_STREETS + 2
_TAB_ROWS = 8

_R = 256


def _body(sidx_ref, tab_ref, idxs_ref, emb_ref):
    sidx = sidx_ref[...]
    r, lcur = sidx.shape
    fill = jnp.full((r, _MAX_SEQ_LEN - lcur), _PAD_TOKEN, dtype=sidx.dtype)
    idxs = jnp.concatenate([sidx, fill], axis=1)
    idxs_ref[...] = idxs
    tab = tab_ref[...]
    classes = jax.lax.broadcasted_iota(jnp.int32, (1, 1, _TAB_ROWS), 2)
    onehot = (idxs[..., None] == classes).astype(jnp.float32)
    emb = jax.lax.dot_general(
        onehot.reshape(r * _MAX_SEQ_LEN, _TAB_ROWS), tab,
        (((1,), (0,)), ((), ())), preferred_element_type=jnp.float32)
    emb_ref[...] = emb.reshape(r, _MAX_SEQ_LEN, _EMBED_DIM)


def kernel(street_idxs, table):
    b, lcur = street_idxs.shape
    tab8 = jnp.concatenate(
        [table, jnp.zeros((_TAB_ROWS - _VOCAB, _EMBED_DIM), table.dtype)], axis=0)
    grid = (b // _R,)
    idxs, emb = pl.pallas_call(
        _body,
        grid=grid,
        in_specs=[
            pl.BlockSpec((_R, lcur), lambda i: (i, 0)),
            pl.BlockSpec((_TAB_ROWS, _EMBED_DIM), lambda i: (0, 0)),
        ],
        out_specs=[
            pl.BlockSpec((_R, _MAX_SEQ_LEN), lambda i: (i, 0)),
            pl.BlockSpec((_R, _MAX_SEQ_LEN, _EMBED_DIM), lambda i: (i, 0, 0)),
        ],
        out_shape=[
            jax.ShapeDtypeStruct((b, _MAX_SEQ_LEN), street_idxs.dtype),
            jax.ShapeDtypeStruct((b, _MAX_SEQ_LEN, _EMBED_DIM), jnp.float32),
        ],
    )(street_idxs, tab8)
    return (idxs, emb)
